# baseline (speedup 1.0000x reference)
import functools

import jax
import jax.numpy as jnp
from jax import lax
from jax.experimental import pallas as pl
from jax.experimental.pallas import tpu as pltpu
from jax.experimental.pallas import tpu_sc as plsc

B = 16384
NCAT = 26
VOCAB = 100000
D = 32
NCONT = 13
NBINS = 8
NTOK = NCAT + NCONT

NC, NS = 2, 16
NW = NC * NS
ROWS = B * NCAT
ROWS_PER_W = ROWS // NW
CHUNK = 128
STEPS = ROWS_PER_W // CHUNK


def _gather_body(table_h, idx_h, dst_h, out_h, idx_v, dst_v, buf_v,
                 gsem, wsem):
    wid = lax.axis_index("s") * NC + lax.axis_index("c")
    pltpu.sync_copy(idx_h.at[wid], idx_v)
    pltpu.sync_copy(dst_h.at[wid], dst_v)

    def step(j, carry):
        pltpu.async_copy(table_h.at[idx_v.at[j]], buf_v, gsem).wait()
        pltpu.async_copy(buf_v, out_h.at[dst_v.at[j]], wsem).wait()
        return carry

    lax.fori_loop(0, STEPS, step, 0)


@functools.cache
def _sc_gather():
    return pl.kernel(
        _gather_body,
        out_type=jax.ShapeDtypeStruct((B * NTOK, D), jnp.float32),
        mesh=plsc.VectorSubcoreMesh(core_axis_name="c", subcore_axis_name="s",
                                    num_cores=NC, num_subcores=NS),
        compiler_params=pltpu.CompilerParams(use_tc_tiling_on_sc=False),
        scratch_types=[
            pltpu.VMEM((STEPS, CHUNK), jnp.int32),
            pltpu.VMEM((STEPS, CHUNK), jnp.int32),
            pltpu.VMEM((CHUNK, D), jnp.float32),
            pltpu.SemaphoreType.DMA,
            pltpu.SemaphoreType.DMA,
        ],
    )


def _plr_kernel(o_in_ref, xrep_ref, bb_ref, w_ref, b_ref, o_ref):
    enc = jnp.maximum(1.0 - jnp.abs(xrep_ref[...] - bb_ref[...]), 0.0)
    o_ref[:, 0, 0, :] = (
        jnp.dot(enc, w_ref[...], preferred_element_type=jnp.float32)
        + b_ref[...]
    )


def _plr_into(out4d, x_rep, bb_flat, W, b):
    blk = 2048
    ncols = NCONT * D
    return pl.pallas_call(
        _plr_kernel,
        grid=(B // blk,),
        in_specs=[
            pl.BlockSpec((blk, 1, 1, ncols), lambda i: (i, 2, 0, 0)),
            pl.BlockSpec((blk, NCONT * NBINS), lambda i: (i, 0)),
            pl.BlockSpec((1, NCONT * NBINS), lambda i: (0, 0)),
            pl.BlockSpec((NCONT * NBINS, ncols), lambda i: (0, 0)),
            pl.BlockSpec((1, ncols), lambda i: (0, 0)),
        ],
        out_specs=pl.BlockSpec((blk, 1, 1, ncols), lambda i: (i, 2, 0, 0)),
        out_shape=jax.ShapeDtypeStruct((B, 3, 1, ncols), jnp.float32),
        input_output_aliases={0: 0},
    )(out4d, x_rep, bb_flat, W, b)


def kernel(x_cat, x_cont, tables, bin_boundaries, W, b):
    table_flat = tables.reshape(NCAT * VOCAB, D)
    idx_flat = (x_cat.astype(jnp.int32)
                + (jnp.arange(NCAT, dtype=jnp.int32) * VOCAB)[None, :])
    idx3 = idx_flat.reshape(NW, STEPS, CHUNK)
    p = jnp.arange(ROWS, dtype=jnp.int32)
    dst3 = ((p // NCAT) * NTOK + (p % NCAT)).reshape(NW, STEPS, CHUNK)
    out_flat = _sc_gather()(table_flat, idx3, dst3)

    x_rep = jnp.repeat(x_cont, NBINS, axis=1)
    bb_flat = bin_boundaries.reshape(1, NCONT * NBINS)
    out4d = _plr_into(out_flat.reshape(B, 3, 1, NCONT * D), x_rep, bb_flat,
                      W, b.reshape(1, NCONT * D))
    return out4d.reshape(B, NTOK, D)


# speedup vs baseline: 1.1070x; 1.1070x over previous
import functools

import jax
import jax.numpy as jnp
from jax import lax
from jax.experimental import pallas as pl
from jax.experimental.pallas import tpu as pltpu
from jax.experimental.pallas import tpu_sc as plsc

B = 16384
NCAT = 26
VOCAB = 100000
D = 32
NCONT = 13
NBINS = 8
NTOK = NCAT + NCONT

NC, NS = 2, 16
NW = NC * NS
ROWS = B * NCAT
ROWS_PER_W = ROWS // NW
CHUNK = 128
STEPS = ROWS_PER_W // CHUNK
CROWS = B * NCONT
CROWS_PER_W = CROWS // NW
CSTEPS = CROWS_PER_W // CHUNK


def _assemble_body(table_h, idx_h, dst_h, cdst_h, cont_h, out_h,
                   idx_v, dst_v, cdst_v, buf_v, gsem, wsem):
    wid = lax.axis_index("s") * NC + lax.axis_index("c")
    pltpu.sync_copy(idx_h.at[wid], idx_v)
    pltpu.sync_copy(dst_h.at[wid], dst_v)
    pltpu.sync_copy(cdst_h.at[wid], cdst_v)
    cbase = wid * CROWS_PER_W

    def cat_step(j, carry):
        pltpu.async_copy(table_h.at[idx_v.at[j]], buf_v, gsem).wait()
        pltpu.async_copy(buf_v, out_h.at[dst_v.at[j]], wsem).wait()
        return carry

    def cont_step(j, carry):
        pltpu.async_copy(cont_h.at[pl.ds(cbase + j * CHUNK, CHUNK)],
                         buf_v, gsem).wait()
        pltpu.async_copy(buf_v, out_h.at[cdst_v.at[j]], wsem).wait()
        return carry

    lax.fori_loop(0, STEPS, cat_step, 0)
    lax.fori_loop(0, CSTEPS, cont_step, 0)


@functools.cache
def _sc_assemble():
    return pl.kernel(
        _assemble_body,
        out_type=jax.ShapeDtypeStruct((B * NTOK, D), jnp.float32),
        mesh=plsc.VectorSubcoreMesh(core_axis_name="c", subcore_axis_name="s",
                                    num_cores=NC, num_subcores=NS),
        compiler_params=pltpu.CompilerParams(use_tc_tiling_on_sc=False),
        scratch_types=[
            pltpu.VMEM((STEPS, CHUNK), jnp.int32),
            pltpu.VMEM((STEPS, CHUNK), jnp.int32),
            pltpu.VMEM((CSTEPS, CHUNK), jnp.int32),
            pltpu.VMEM((CHUNK, D), jnp.float32),
            pltpu.SemaphoreType.DMA,
            pltpu.SemaphoreType.DMA,
        ],
    )


def _plr_kernel(xrep_ref, bb_ref, w_ref, b_ref, o_ref):
    enc = jnp.maximum(1.0 - jnp.abs(xrep_ref[...] - bb_ref[...]), 0.0)
    o_ref[...] = (
        jnp.dot(enc, w_ref[...], preferred_element_type=jnp.float32)
        + b_ref[...]
    )


def _plr(x_rep, bb_flat, W, b):
    blk = 2048
    ncols = NCONT * D
    return pl.pallas_call(
        _plr_kernel,
        grid=(B // blk,),
        in_specs=[
            pl.BlockSpec((blk, NCONT * NBINS), lambda i: (i, 0)),
            pl.BlockSpec((1, NCONT * NBINS), lambda i: (0, 0)),
            pl.BlockSpec((NCONT * NBINS, ncols), lambda i: (0, 0)),
            pl.BlockSpec((1, ncols), lambda i: (0, 0)),
        ],
        out_specs=pl.BlockSpec((blk, ncols), lambda i: (i, 0)),
        out_shape=jax.ShapeDtypeStruct((B, ncols), jnp.float32),
    )(x_rep, bb_flat, W, b)


def kernel(x_cat, x_cont, tables, bin_boundaries, W, b):
    x_rep = jnp.repeat(x_cont, NBINS, axis=1)
    bb_flat = bin_boundaries.reshape(1, NCONT * NBINS)
    cont = _plr(x_rep, bb_flat, W, b.reshape(1, NCONT * D))

    table_flat = tables.reshape(NCAT * VOCAB, D)
    idx_flat = (x_cat.astype(jnp.int32)
                + (jnp.arange(NCAT, dtype=jnp.int32) * VOCAB)[None, :])
    idx3 = idx_flat.reshape(NW, STEPS, CHUNK)
    p = jnp.arange(ROWS, dtype=jnp.int32)
    dst3 = ((p // NCAT) * NTOK + (p % NCAT)).reshape(NW, STEPS, CHUNK)
    q = jnp.arange(CROWS, dtype=jnp.int32)
    cdst3 = ((q // NCONT) * NTOK + NCAT + (q % NCONT)).reshape(
        NW, CSTEPS, CHUNK)

    out_flat = _sc_assemble()(table_flat, idx3, dst3, cdst3,
                              cont.reshape(CROWS, D))
    return out_flat.reshape(B, NTOK, D)
